# baseline (speedup 1.0000x reference)
import functools

import jax
import jax.numpy as jnp
import numpy as np
from jax import lax
from jax.experimental import pallas as pl
from jax.experimental.pallas import tpu as pltpu
from jax.experimental.pallas import tpu_sc as plsc

_B, _C, _H, _W = 4, 384, 64, 64
_HW = _H * _W
_COUNTS = 1024
_K = 9
_NPLANE = 4 * _K
_OUT_PER_IMG = _COUNTS * _NPLANE
_NBLK = _HW // 128
_NCHUNK = _COUNTS // 128

_s_np, _r_np = np.meshgrid(np.array([32.0, 64.0, 128.0], np.float32),
                           np.array([0.5, 1.0, 2.0], np.float32))
_sf = _s_np.flatten().astype(np.float32)
_rf = _r_np.flatten().astype(np.float32)
_heights = (_sf / np.sqrt(_rf)).astype(np.float32)
_widths = (_sf * np.sqrt(_rf)).astype(np.float32)
_BH2 = [float(np.float32(0.5) * h) for h in _heights]
_BW2 = [float(np.float32(0.5) * w) for w in _widths]
_INV_NORM = 1.0 / 512.0


def _hm_body(x_ref, heat_ref):
    x = x_ref[0]
    m1 = jnp.mean(x, axis=1, keepdims=True)
    m = jnp.mean(m1, axis=2, keepdims=True)
    heat_ref[0] = jnp.sum(jnp.abs(x - m), axis=0)


_hm_call = pl.pallas_call(
    _hm_body,
    grid=(_B,),
    in_specs=[pl.BlockSpec((1, _C, _H, _W), lambda i: (i, 0, 0, 0))],
    out_specs=pl.BlockSpec((1, _H, _W), lambda i: (i, 0, 0)),
    out_shape=jax.ShapeDtypeStruct((_B, _H, _W), jnp.float32),
)


def _sel_body(heat_ref, sel_ref):
    heat = heat_ref[0]
    hi = lax.bitcast_convert_type(heat, jnp.int32)

    def bit_step(i, pfx):
        trial = pfx | lax.shift_left(jnp.int32(1), jnp.int32(30) - i)
        cnt = jnp.sum((hi >= trial).astype(jnp.int32))
        return jnp.where(cnt >= _COUNTS, trial, pfx)

    pfx = lax.fori_loop(0, 31, bit_step, jnp.int32(0))
    thr = lax.bitcast_convert_type(pfx, jnp.float32)

    mask = (heat >= thr).astype(jnp.float32)

    r128 = lax.broadcasted_iota(jnp.int32, (128, 128), 0)
    c128 = lax.broadcasted_iota(jnp.int32, (128, 128), 1)
    ut128 = (r128 <= c128).astype(jnp.float32)
    parts = []
    run = jnp.zeros((1, 1), jnp.float32)
    for blk in range(_NBLK):
        mblk = mask[:, blk * 128:(blk + 1) * 128]
        pref = jnp.dot(mblk, ut128, preferred_element_type=jnp.float32)
        parts.append(pref + run)
        run = run + pref[:, 127:128]
    rank = jnp.concatenate(parts, axis=1)
    srank = rank * mask

    fi = lax.broadcasted_iota(jnp.int32, (_HW, 1), 0)
    hi_col = (fi // 64).astype(jnp.float32)
    lo_col = (fi % 64).astype(jnp.float32)
    prow = lax.broadcasted_iota(jnp.int32, (128, 1), 0)
    srank_b = jnp.broadcast_to(srank, (128, _HW))
    for q in range(_NCHUNK):
        target = (prow + (q * 128 + 1)).astype(jnp.float32)
        onehot = (srank_b == target).astype(jnp.float32)
        s_hi = jnp.dot(onehot, hi_col, preferred_element_type=jnp.float32)
        s_lo = jnp.dot(onehot, lo_col, preferred_element_type=jnp.float32)
        sel_ref[0, q] = s_hi[:, 0] * 64.0 + s_lo[:, 0]


_sel_call = pl.pallas_call(
    _sel_body,
    grid=(_B,),
    in_specs=[pl.BlockSpec((1, 1, _HW), lambda i: (i, 0, 0))],
    out_specs=pl.BlockSpec((1, _NCHUNK, 128), lambda i: (i, 0, 0)),
    out_shape=jax.ShapeDtypeStruct((_B, _NCHUNK, 128), jnp.float32),
)

_NC, _NS = 2, 16


def _sc_body(sel_hbm, out_hbm, sel_v, box_v):
    wid = lax.axis_index("s") * _NC + lax.axis_index("c")

    @pl.when(wid < _B)
    def _():
        b = wid
        pltpu.sync_copy(sel_hbm.at[pl.ds(b * _COUNTS, _COUNTS)], sel_v)

        def box_step(t, carry):
            iv = sel_v[pl.ds(t * 16, 16)].astype(jnp.int32)
            py = lax.shift_right_logical(iv, 6)
            px = iv & 63
            cy = py.astype(jnp.float32) * 0.125
            cx = px.astype(jnp.float32) * 0.125
            for k in range(_K):
                y1 = jnp.clip((cy - _BH2[k]) * _INV_NORM, 0.0, 1.0)
                x1 = jnp.clip((cx - _BW2[k]) * _INV_NORM, 0.0, 1.0)
                y2 = jnp.clip((cy + _BH2[k]) * _INV_NORM, 0.0, 1.0)
                x2 = jnp.clip((cx + _BW2[k]) * _INV_NORM, 0.0, 1.0)
                box_v[pl.ds((4 * k + 0) * _COUNTS + t * 16, 16)] = y1
                box_v[pl.ds((4 * k + 1) * _COUNTS + t * 16, 16)] = x1
                box_v[pl.ds((4 * k + 2) * _COUNTS + t * 16, 16)] = y2
                box_v[pl.ds((4 * k + 3) * _COUNTS + t * 16, 16)] = x2
            return carry

        lax.fori_loop(0, _COUNTS // 16, box_step, jnp.int32(0))
        pltpu.sync_copy(box_v, out_hbm.at[pl.ds(b * _OUT_PER_IMG, _OUT_PER_IMG)])


def _make_sc_call():
    return functools.partial(
        pl.kernel,
        out_type=jax.ShapeDtypeStruct((_B * _OUT_PER_IMG,), jnp.float32),
        mesh=plsc.VectorSubcoreMesh(core_axis_name="c", subcore_axis_name="s"),
        scratch_types=[
            pltpu.VMEM((_COUNTS,), jnp.float32),
            pltpu.VMEM((_OUT_PER_IMG,), jnp.float32),
        ],
    )(_sc_body)


def _tr_body(planes_ref, out_ref):
    out_ref[0] = planes_ref[0].T


_tr_call = pl.pallas_call(
    _tr_body,
    grid=(_B,),
    in_specs=[pl.BlockSpec((1, _NPLANE, _COUNTS), lambda i: (i, 0, 0))],
    out_specs=pl.BlockSpec((1, _COUNTS, _NPLANE), lambda i: (i, 0, 0)),
    out_shape=jax.ShapeDtypeStruct((_B, _COUNTS, _NPLANE), jnp.float32),
)


def kernel(feature_maps):
    x = feature_maps[0]
    heat = _hm_call(x)
    sel = _sel_call(heat.reshape(_B, 1, _HW))
    planes = _make_sc_call()(sel.reshape(-1))
    out = _tr_call(planes.reshape(_B, _NPLANE, _COUNTS))
    return out.reshape(_B, _COUNTS * _K, 4)
